# baseline (speedup 1.0000x reference)
import functools

import jax
import jax.numpy as jnp
from jax import lax
from jax.experimental import pallas as pl
from jax.experimental.pallas import tpu as pltpu
from jax.experimental.pallas import tpu_sc as plsc

_NC = 2
_NS = 16
_NW = _NC * _NS

_UNTILED = pltpu.CompilerParams(use_tc_tiling_on_sc=False,
                                needs_layout_passes=False)


def _mesh():
    return plsc.VectorSubcoreMesh(core_axis_name="c", subcore_axis_name="s",
                                  num_cores=_NC)


def _wid():
    return lax.axis_index("s") * _NC + lax.axis_index("c")


def _sc_gather_body(tab_hbm, idx_hbm, out_hbm, idx_v, w_v, sem):
    w = _wid()
    pltpu.sync_copy(idx_hbm.at[w], idx_v)
    pltpu.async_copy(tab_hbm.at[idx_v], w_v, sem).wait()
    pltpu.sync_copy(w_v, out_hbm.at[w])


def _sc_gather(table_flat, idxf):
    n = idxf.shape[0]
    per_w = n // _NW
    k = pl.kernel(
        _sc_gather_body,
        out_type=jax.ShapeDtypeStruct((_NW, per_w), jnp.float32),
        mesh=_mesh(),
        scratch_types=[
            pltpu.VMEM((per_w,), jnp.int32),
            pltpu.VMEM((per_w,), jnp.float32),
            pltpu.SemaphoreType.DMA,
        ],
        compiler_params=_UNTILED,
    )
    return k(table_flat, idxf.reshape(_NW, per_w))


def _dense_body(g_ref, wa_ref, ba_ref, wp_ref, wo_ref, bo_ref, o_ref):
    e = g_ref[...]
    wa = wa_ref[...]
    ba = ba_ref[...]
    wp = wp_ref[...]
    wo = wo_ref[...]
    scores = []
    qs = []
    for (i, j) in ((0, 1), (0, 2), (1, 2)):
        p = e[:, 3 * i:3 * i + 3] * e[:, 3 * j:3 * j + 3]
        h = (p[:, 0:1] * wa[0:1, :] + p[:, 1:2] * wa[1:2, :]
             + p[:, 2:3] * wa[2:3, :] + ba)
        h = jnp.maximum(h, 0.0)
        scores.append(jnp.sum(h * wp, axis=1, keepdims=True))
        qs.append(jnp.sum(p * wo, axis=1, keepdims=True))
    m = jnp.maximum(jnp.maximum(scores[0], scores[1]), scores[2])
    es = [jnp.exp(s - m) for s in scores]
    z = es[0] + es[1] + es[2]
    o_ref[...] = (es[0] * qs[0] + es[1] * qs[1] + es[2] * qs[2]) / z \
        + bo_ref[0, 0]


def _tc_dense(g, W_attn, b_attn, W_proj, W_out, b_out):
    b = g.shape[0]
    s = 2048
    rep = lambda i: (0, 0)
    return pl.pallas_call(
        _dense_body,
        grid=(b // s,),
        in_specs=[
            pl.BlockSpec((s, 9), lambda i: (i, 0)),
            pl.BlockSpec((3, 64), rep),
            pl.BlockSpec((1, 64), rep),
            pl.BlockSpec((1, 64), rep),
            pl.BlockSpec((1, 3), rep),
            pl.BlockSpec((1, 1), rep),
        ],
        out_specs=pl.BlockSpec((s, 1), lambda i: (i, 0)),
        out_shape=jax.ShapeDtypeStruct((b, 1), jnp.float32),
    )(g, W_attn, b_attn.reshape(1, -1), W_proj.reshape(1, -1),
      W_out.reshape(1, -1), b_out.reshape(1, 1))


def kernel(inputs, table, W_attn, b_attn, W_proj, W_out, b_out):
    bsz = inputs.shape[0]
    idx = inputs.astype(jnp.int32).reshape(-1)
    idxf = (idx[:, None] * 3
            + jnp.arange(3, dtype=jnp.int32)[None, :]).reshape(-1)
    one = 1.0 + 0.0 * b_out[0]
    tabf = table.reshape(-1) * one
    words = _sc_gather(tabf, idxf)
    g = words.reshape(bsz, 9)
    return _tc_dense(g, W_attn, b_attn, W_proj, W_out, b_out)


# speedup vs baseline: 24.6132x; 24.6132x over previous
import functools

import jax
import jax.numpy as jnp
from jax import lax
from jax.experimental import pallas as pl
from jax.experimental.pallas import tpu as pltpu
from jax.experimental.pallas import tpu_sc as plsc

_NC = 2
_NS = 16
_NW = _NC * _NS

_UNTILED = pltpu.CompilerParams(use_tc_tiling_on_sc=False,
                                needs_layout_passes=False)


def _mesh():
    return plsc.VectorSubcoreMesh(core_axis_name="c", subcore_axis_name="s",
                                  num_cores=_NC)


def _wid():
    return lax.axis_index("s") * _NC + lax.axis_index("c")


def _sc_gather_body(tab_hbm, idx_hbm, out_hbm, idx_v, w_v, sem):
    w = _wid()
    pltpu.sync_copy(idx_hbm.at[w], idx_v)
    pltpu.async_copy(tab_hbm.at[idx_v], w_v, sem).wait()
    pltpu.sync_copy(w_v, out_hbm.at[w])


def _sc_gather(table_flat, idxf):
    n = idxf.shape[0]
    per_w = n // _NW
    k = pl.kernel(
        _sc_gather_body,
        out_type=jax.ShapeDtypeStruct((_NW, per_w), jnp.float32),
        mesh=_mesh(),
        scratch_types=[
            pltpu.VMEM((per_w,), jnp.int32),
            pltpu.VMEM((per_w,), jnp.float32),
            pltpu.SemaphoreType.DMA,
        ],
        compiler_params=_UNTILED,
    )
    return k(table_flat, idxf.reshape(_NW, per_w))


def _dense_body(g_ref, wa_ref, ba_ref, wp_ref, wo_ref, bo_ref, o_ref):
    e = g_ref[...]
    wa = wa_ref[...]
    ba = ba_ref[...]
    wp = wp_ref[...]
    wo = wo_ref[...]
    scores = []
    qs = []
    for (i, j) in ((0, 1), (0, 2), (1, 2)):
        p = e[:, 3 * i:3 * i + 3] * e[:, 3 * j:3 * j + 3]
        h = (p[:, 0:1] * wa[0:1, :] + p[:, 1:2] * wa[1:2, :]
             + p[:, 2:3] * wa[2:3, :] + ba)
        h = jnp.maximum(h, 0.0)
        scores.append(jnp.sum(h * wp, axis=1, keepdims=True))
        qs.append(jnp.sum(p * wo, axis=1, keepdims=True))
    m = jnp.maximum(jnp.maximum(scores[0], scores[1]), scores[2])
    es = [jnp.exp(s - m) for s in scores]
    z = es[0] + es[1] + es[2]
    o_ref[...] = (es[0] * qs[0] + es[1] * qs[1] + es[2] * qs[2]) / z \
        + bo_ref[0, 0]


def _tc_dense(g, W_attn, b_attn, W_proj, W_out, b_out):
    b = g.shape[0]
    s = 2048
    rep = lambda i: (0, 0)
    return pl.pallas_call(
        _dense_body,
        grid=(b // s,),
        in_specs=[
            pl.BlockSpec((s, 9), lambda i: (i, 0)),
            pl.BlockSpec((3, 64), rep),
            pl.BlockSpec((1, 64), rep),
            pl.BlockSpec((1, 64), rep),
            pl.BlockSpec((1, 3), rep),
            pl.BlockSpec((1, 1), rep),
        ],
        out_specs=pl.BlockSpec((s, 1), lambda i: (i, 0)),
        out_shape=jax.ShapeDtypeStruct((b, 1), jnp.float32),
    )(g, W_attn, b_attn.reshape(1, -1), W_proj.reshape(1, -1),
      W_out.reshape(1, -1), b_out.reshape(1, 1))


def kernel(inputs, table, W_attn, b_attn, W_proj, W_out, b_out):
    bsz = inputs.shape[0]
    idx = inputs.astype(jnp.int32).reshape(-1)
    v = table.shape[0]
    idxf = (jnp.arange(3, dtype=jnp.int32)[None, :] * v
            + idx[:, None]).reshape(-1)
    tabf = table.T.reshape(-1)
    words = _sc_gather(tabf, idxf)
    g = words.reshape(bsz, 9)
    return _tc_dense(g, W_attn, b_attn, W_proj, W_out, b_out)


# speedup vs baseline: 24.6495x; 1.0015x over previous
import jax
import jax.numpy as jnp
from jax import lax
from jax.experimental import pallas as pl
from jax.experimental.pallas import tpu as pltpu
from jax.experimental.pallas import tpu_sc as plsc

_NC = 2
_NS = 16
_NW = _NC * _NS

_UNTILED = pltpu.CompilerParams(use_tc_tiling_on_sc=False,
                                needs_layout_passes=False)


def _mesh():
    return plsc.VectorSubcoreMesh(core_axis_name="c", subcore_axis_name="s",
                                  num_cores=_NC)


def _wid():
    return lax.axis_index("s") * _NC + lax.axis_index("c")


def _sc_gather_body(tab_hbm, idx_hbm, out_hbm, idx_v, w_v, sem):
    w = _wid()
    pltpu.sync_copy(idx_hbm.at[w], idx_v)
    pltpu.async_copy(tab_hbm.at[idx_v], w_v, sem).wait()
    pltpu.sync_copy(w_v, out_hbm.at[w])


def _sc_gather(table_flat, idxf):
    n = idxf.shape[0]
    per_w = n // _NW
    k = pl.kernel(
        _sc_gather_body,
        out_type=jax.ShapeDtypeStruct((_NW, per_w), jnp.float32),
        mesh=_mesh(),
        scratch_types=[
            pltpu.VMEM((per_w,), jnp.int32),
            pltpu.VMEM((per_w,), jnp.float32),
            pltpu.SemaphoreType.DMA,
        ],
        compiler_params=_UNTILED,
    )
    return k(table_flat, idxf.reshape(_NW, per_w))


def _dense_body(g_ref, wa_ref, ba_ref, wp_ref, wo_ref, bo_ref, o_ref):
    e = g_ref[...]
    wa = wa_ref[...]
    ba = ba_ref[...]
    wp = wp_ref[...]
    wo = wo_ref[...]
    scores = []
    qs = []
    for (i, j) in ((0, 1), (0, 2), (1, 2)):
        p = e[:, 3 * i:3 * i + 3] * e[:, 3 * j:3 * j + 3]
        h = (p[:, 0:1] * wa[0:1, :] + p[:, 1:2] * wa[1:2, :]
             + p[:, 2:3] * wa[2:3, :] + ba)
        h = jnp.maximum(h, 0.0)
        scores.append(jnp.sum(h * wp, axis=1, keepdims=True))
        qs.append(jnp.sum(p * wo, axis=1, keepdims=True))
    m = jnp.maximum(jnp.maximum(scores[0], scores[1]), scores[2])
    es = [jnp.exp(s - m) for s in scores]
    z = es[0] + es[1] + es[2]
    o_ref[...] = (es[0] * qs[0] + es[1] * qs[1] + es[2] * qs[2]) / z \
        + bo_ref[0, 0]


def _tc_dense(g, W_attn, b_attn, W_proj, W_out, b_out):
    b = g.shape[0]
    s = 2048
    rep = lambda i: (0, 0)
    return pl.pallas_call(
        _dense_body,
        grid=(b // s,),
        in_specs=[
            pl.BlockSpec((s, 9), lambda i: (i, 0)),
            pl.BlockSpec((3, 64), rep),
            pl.BlockSpec((1, 64), rep),
            pl.BlockSpec((1, 64), rep),
            pl.BlockSpec((1, 3), rep),
            pl.BlockSpec((1, 1), rep),
        ],
        out_specs=pl.BlockSpec((s, 1), lambda i: (i, 0)),
        out_shape=jax.ShapeDtypeStruct((b, 1), jnp.float32),
    )(g, W_attn, b_attn.reshape(1, -1), W_proj.reshape(1, -1),
      W_out.reshape(1, -1), b_out.reshape(1, 1))


def kernel(inputs, table, W_attn, b_attn, W_proj, W_out, b_out):
    bsz = inputs.shape[0]
    idx = inputs.astype(jnp.int32).reshape(-1)
    v = table.shape[0]
    idxf = (jnp.arange(3, dtype=jnp.int32)[None, :] * v
            + idx[:, None]).reshape(-1)
    tabf = table.T.reshape(-1)
    words = _sc_gather(tabf, idxf)
    g = words.reshape(bsz, 9)
    return _tc_dense(g, W_attn, b_attn, W_proj, W_out, b_out)
